# baseline (speedup 1.0000x reference)
import functools

import jax
import jax.numpy as jnp
from jax import lax
from jax.experimental import pallas as pl
from jax.experimental.pallas import tpu as pltpu
from jax.experimental.pallas import tpu_sc as plsc

E = 8
TOP_K = 2
D_MODEL = 2048
D_FF = 1024
T = 4096
P = T * TOP_K
BM = 256
C = P + E * BM
NT = C // BM
NTE = 48
L = 16

D_TILES = 16
TOK_PER = T // D_TILES
PAIR_PER = 2 * TOK_PER
NW = 32


def _iota():
    return lax.iota(jnp.int32, L)


def _splat(val):
    return jnp.full((L,), val, jnp.int32)


def _bcast_lane(vec, lane):
    zero = jnp.zeros((), vec.dtype)
    return jnp.full((L,), jnp.sum(jnp.where(_iota() == lane, vec, zero)))


def _route_body(rl_ref, cnt_ref, pe_ref, pw_ref, lg_v, pe_v, pwf_v, row16_v):
    s = lax.axis_index("s")
    iot = _iota()

    pltpu.sync_copy(rl_ref.at[pl.ds(s * TOK_PER * E, TOK_PER * E)], lg_v)
    cnt = [jnp.zeros((L,), jnp.int32) for _ in range(E)]
    minf = jnp.full((L,), -jnp.inf, jnp.float32)
    for g in range(TOK_PER // L):
        base = g * L * E
        vals = [plsc.load_gather(lg_v, [base + iot * E + e]) for e in range(E)]
        m1 = vals[0]
        for e in range(1, E):
            m1 = jnp.maximum(m1, vals[e])
        idx1 = _splat(E)
        for e in range(E):
            idx1 = jnp.minimum(idx1, jnp.where(vals[e] == m1, e, E))
        m2 = minf
        for e in range(E):
            v2 = jnp.where(idx1 == e, minf, vals[e])
            m2 = jnp.maximum(m2, v2)
        idx2 = _splat(E)
        for e in range(E):
            v2 = jnp.where(idx1 == e, minf, vals[e])
            idx2 = jnp.minimum(idx2, jnp.where(v2 == m2, e, E))
        w1 = 1.0 / (1.0 + jnp.exp(m2 - m1))
        w2 = 1.0 - w1
        pbase = 2 * g * L
        plsc.store_scatter(pe_v, [pbase + 2 * iot], idx1)
        plsc.store_scatter(pe_v, [pbase + 2 * iot + 1], idx2)
        plsc.store_scatter(pwf_v, [pbase + 2 * iot], w1)
        plsc.store_scatter(pwf_v, [pbase + 2 * iot + 1], w2)
        for e in range(E):
            cnt[e] = cnt[e] + plsc.all_reduce_population_count(idx1 == e)
            cnt[e] = cnt[e] + plsc.all_reduce_population_count(idx2 == e)

    cnt_vec = jnp.zeros((L,), jnp.int32)
    for e in range(E):
        cnt_vec = cnt_vec + jnp.where(iot == e, cnt[e], 0)
    row16_v[...] = cnt_vec
    pltpu.sync_copy(row16_v, cnt_ref.at[s])
    pltpu.sync_copy(pe_v, pe_ref.at[s])
    pltpu.sync_copy(pwf_v, pw_ref.at[s])


def _route(rl_flat):
    mesh = plsc.VectorSubcoreMesh(core_axis_name="c", subcore_axis_name="s",
                                  num_cores=1)
    f = pl.kernel(
        _route_body,
        compiler_params=pltpu.CompilerParams(needs_layout_passes=False),
        out_type=(
            jax.ShapeDtypeStruct((D_TILES, L), jnp.int32),
            jax.ShapeDtypeStruct((D_TILES, PAIR_PER), jnp.int32),
            jax.ShapeDtypeStruct((D_TILES, PAIR_PER), jnp.float32),
        ),
        mesh=mesh,
        scratch_types=[
            pltpu.VMEM((TOK_PER * E,), jnp.float32),
            pltpu.VMEM((PAIR_PER,), jnp.int32),
            pltpu.VMEM((PAIR_PER,), jnp.float32),
            pltpu.VMEM((L,), jnp.int32),
        ],
    )
    return f(rl_flat)


def _sort_body(cnt_ref, pe_in_ref, pos_ref, tok_ref, te_ref,
               pe_v, row16_v, pos2_v, tok2_v, cnt2d_v):
    s = lax.axis_index("s")
    iot = _iota()

    pltpu.sync_copy(cnt_ref, cnt2d_v)
    pltpu.sync_copy(pe_in_ref.at[s], pe_v)
    tot = jnp.zeros((L,), jnp.int32)
    pre = jnp.zeros((L,), jnp.int32)
    s_spl = jnp.full((L,), s, jnp.int32)
    for t in range(D_TILES):
        row = cnt2d_v[t]
        pre = pre + jnp.where(_splat(t) < s_spl, row, 0)
        tot = tot + row
    rup = ((tot + (BM - 1)) >> 8) << 8
    off_incl = plsc.cumsum(rup)
    off_al = off_incl - rup
    start_vec = off_al + pre

    @pl.when(s == 0)
    def _():
        ends = [_bcast_lane(off_incl, e) for e in range(E - 1)]
        for c in range(NTE // L):
            pos0 = (c * L + iot) * BM
            te = jnp.zeros((L,), jnp.int32)
            for e in range(E - 1):
                te = te + jnp.where(pos0 >= ends[e], 1, 0)
            row16_v[...] = te
            pltpu.sync_copy(row16_v, te_ref.at[pl.ds(c * L, L)])

    start = [_bcast_lane(start_vec, e) for e in range(E)]
    for g in range(PAIR_PER // L):
        pe_g = pe_v[pl.ds(g * L, L)]
        pos = jnp.zeros((L,), jnp.int32)
        for e in range(E):
            m = pe_g == e
            incl = plsc.cumsum(jnp.where(m, 1, 0))
            pos = jnp.where(m, start[e] + incl - 1, pos)
            start[e] = start[e] + plsc.all_reduce_population_count(m)
        pos2_v[g] = pos
        tok2_v[g] = s_spl * TOK_PER + ((g * L + iot) >> 1)

    pltpu.sync_copy(pos2_v, pos_ref.at[s])
    pltpu.sync_copy(tok2_v, tok_ref.at[s])


def _sort(cnts, pe3):
    mesh = plsc.VectorSubcoreMesh(core_axis_name="c", subcore_axis_name="s",
                                  num_cores=1)
    f = pl.kernel(
        _sort_body,
        compiler_params=pltpu.CompilerParams(needs_layout_passes=False),
        out_type=(
            jax.ShapeDtypeStruct((D_TILES, PAIR_PER // L, L), jnp.int32),
            jax.ShapeDtypeStruct((D_TILES, PAIR_PER // L, L), jnp.int32),
            jax.ShapeDtypeStruct((NTE,), jnp.int32),
        ),
        mesh=mesh,
        scratch_types=[
            pltpu.VMEM((PAIR_PER,), jnp.int32),
            pltpu.VMEM((L,), jnp.int32),
            pltpu.VMEM((PAIR_PER // L, L), jnp.int32),
            pltpu.VMEM((PAIR_PER // L, L), jnp.int32),
            pltpu.VMEM((D_TILES, L), jnp.int32),
        ],
    )
    return f(cnts, pe3)


def _dispatch(rl_flat):
    cnts, pe3, pw3 = _route(rl_flat)
    pos, tok, te = _sort(cnts, pe3)
    return pos, tok, pw3.reshape(D_TILES, PAIR_PER // L, L), te


K2_CH = 8
K2_B = (P // NW) // K2_CH


def _shuffle_body(xb_ref, tok_ref, pos_ref, xs_ref, tokv, posv, buf, sem):
    wid = lax.axis_index("s") * 2 + lax.axis_index("c")
    pltpu.sync_copy(tok_ref.at[wid], tokv)
    pltpu.sync_copy(pos_ref.at[wid], posv)
    for c in range(K2_CH):
        pltpu.async_copy(xb_ref.at[tokv.at[c]], buf, sem).wait()
        pltpu.async_copy(buf, xs_ref.at[posv.at[c]], sem).wait()


D_I32 = D_MODEL // 2


def _shuffle(xb3, tok3, pos3):
    mesh = plsc.VectorSubcoreMesh(core_axis_name="c", subcore_axis_name="s")
    f = pl.kernel(
        _shuffle_body,
        compiler_params=pltpu.CompilerParams(needs_layout_passes=False),
        out_type=jax.ShapeDtypeStruct((C, D_I32 // 128, 128), jnp.int32),
        mesh=mesh,
        scratch_types=[
            pltpu.VMEM((K2_CH, K2_B), jnp.int32),
            pltpu.VMEM((K2_CH, K2_B), jnp.int32),
            pltpu.VMEM((K2_B, D_I32 // 128, 128), jnp.int32),
            pltpu.SemaphoreType.DMA,
        ],
    )
    return f(xb3, tok3, pos3)


def _gmm_body(te_ref, xs_ref, w13_ref, w2_ref, y_ref):
    gu = jnp.dot(xs_ref[...], w13_ref[0], preferred_element_type=jnp.float32)
    gate = gu[:, :D_FF]
    up = gu[:, D_FF:]
    h = (gate * jax.nn.sigmoid(gate)) * up
    y_ref[...] = jnp.dot(h.astype(jnp.bfloat16), w2_ref[0],
                         preferred_element_type=jnp.float32)


def _gmm(te, xs2, w13b, w2b):
    grid_spec = pltpu.PrefetchScalarGridSpec(
        num_scalar_prefetch=1,
        grid=(NT,),
        in_specs=[
            pl.BlockSpec((BM, D_MODEL), lambda t, te: (t, 0)),
            pl.BlockSpec((1, D_MODEL, 2 * D_FF), lambda t, te: (te[t], 0, 0)),
            pl.BlockSpec((1, D_FF, D_MODEL), lambda t, te: (te[t], 0, 0)),
        ],
        out_specs=pl.BlockSpec((BM, D_MODEL), lambda t, te: (t, 0)),
    )
    return pl.pallas_call(
        _gmm_body,
        grid_spec=grid_spec,
        out_shape=jax.ShapeDtypeStruct((C, D_MODEL), jnp.float32),
        compiler_params=pltpu.CompilerParams(
            dimension_semantics=("arbitrary",),
        ),
    )(te, xs2, w13b, w2b)


K4_CH = 8
K4_TOK = (T // NW) // K4_CH
K4_B = 2 * K4_TOK
LC = D_MODEL // L


def _combine_body(y_ref, pos_ref, pw_ref, out_ref, posv, pwv, buf, obuf, sem):
    wid = lax.axis_index("s") * 2 + lax.axis_index("c")
    pltpu.sync_copy(pos_ref.at[wid], posv)
    pltpu.sync_copy(pw_ref.at[wid], pwv)
    tbase = wid * (T // NW)
    for c in range(K4_CH):
        pltpu.async_copy(y_ref.at[posv.at[c]], buf, sem).wait()
        wlo = pwv[c, pl.ds(0, L)]
        whi = pwv[c, pl.ds(L, L)]
        ws = ([_bcast_lane(wlo, i) for i in range(L)]
              + [_bcast_lane(whi, i) for i in range(L)])

        def body(j, _):
            for i in range(K4_TOK):
                a = buf[2 * i, pl.ds(j * L, L)]
                b = buf[2 * i + 1, pl.ds(j * L, L)]
                obuf[i, pl.ds(j * L, L)] = ws[2 * i] * a + ws[2 * i + 1] * b
            return 0

        lax.fori_loop(0, LC, body, 0)
        pltpu.sync_copy(obuf, out_ref.at[pl.ds(tbase + c * K4_TOK, K4_TOK)])


def _combine(y, pos3, pw3):
    mesh = plsc.VectorSubcoreMesh(core_axis_name="c", subcore_axis_name="s")
    f = pl.kernel(
        _combine_body,
        compiler_params=pltpu.CompilerParams(needs_layout_passes=False),
        out_type=jax.ShapeDtypeStruct((T, D_MODEL), jnp.float32),
        mesh=mesh,
        scratch_types=[
            pltpu.VMEM((K4_CH, K4_B), jnp.int32),
            pltpu.VMEM((K4_CH, K4_B), jnp.float32),
            pltpu.VMEM((K4_B, D_MODEL), jnp.float32),
            pltpu.VMEM((K4_TOK, D_MODEL), jnp.float32),
            pltpu.SemaphoreType.DMA,
        ],
    )
    return f(y, pos3, pw3)


def kernel(x, router_logits, w13, w2):
    rl_flat = router_logits.reshape(-1)
    xb_i32 = jax.lax.bitcast_convert_type(
        x.astype(jnp.bfloat16).reshape(T, D_I32, 2), jnp.int32)
    xb3 = xb_i32.reshape(T, D_I32 // 128, 128)
    w13b = w13.astype(jnp.bfloat16)
    w2b = w2.astype(jnp.bfloat16)

    pos, tok, pw, te = _dispatch(rl_flat)
    pos_w = pos.reshape(NW, K2_CH, K2_B)
    tok_w = tok.reshape(NW, K2_CH, K2_B)
    pw_w = pw.reshape(NW, K4_CH, K4_B)

    xs3 = _shuffle(xb3, tok_w, pos_w)
    xs = jax.lax.bitcast_convert_type(
        xs3.reshape(C, D_I32), jnp.bfloat16).reshape(C, D_MODEL)
    y = _gmm(te, xs, w13b, w2b)
    out = _combine(y, pos_w, pw_w)
    return out


# speedup vs baseline: 2.3123x; 2.3123x over previous
import functools

import jax
import jax.numpy as jnp
from jax import lax
from jax.experimental import pallas as pl
from jax.experimental.pallas import tpu as pltpu
from jax.experimental.pallas import tpu_sc as plsc

E = 8
TOP_K = 2
D_MODEL = 2048
D_FF = 1024
T = 4096
P = T * TOP_K
BM = 256
C = P + E * BM
NT = C // BM
NTE = 48
L = 16

D_TILES = 16
TOK_PER = T // D_TILES
PAIR_PER = 2 * TOK_PER
NW = 32


def _iota():
    return lax.iota(jnp.int32, L)


def _splat(val):
    return jnp.full((L,), val, jnp.int32)


def _bcast_lane(vec, lane):
    zero = jnp.zeros((), vec.dtype)
    return jnp.full((L,), jnp.sum(jnp.where(_iota() == lane, vec, zero)))


def _route_body(rl_ref, cnt_ref, pe_ref, pw_ref, lg_v, pe_v, pwf_v, row16_v):
    s = lax.axis_index("s")
    iot = _iota()

    pltpu.sync_copy(rl_ref.at[pl.ds(s * TOK_PER * E, TOK_PER * E)], lg_v)
    cnt = [jnp.zeros((L,), jnp.int32) for _ in range(E)]
    minf = jnp.full((L,), -jnp.inf, jnp.float32)
    for g in range(TOK_PER // L):
        base = g * L * E
        vals = [plsc.load_gather(lg_v, [base + iot * E + e]) for e in range(E)]
        m1 = vals[0]
        for e in range(1, E):
            m1 = jnp.maximum(m1, vals[e])
        idx1 = _splat(E)
        for e in range(E):
            idx1 = jnp.minimum(idx1, jnp.where(vals[e] == m1, e, E))
        m2 = minf
        for e in range(E):
            v2 = jnp.where(idx1 == e, minf, vals[e])
            m2 = jnp.maximum(m2, v2)
        idx2 = _splat(E)
        for e in range(E):
            v2 = jnp.where(idx1 == e, minf, vals[e])
            idx2 = jnp.minimum(idx2, jnp.where(v2 == m2, e, E))
        w1 = 1.0 / (1.0 + jnp.exp(m2 - m1))
        w2 = 1.0 - w1
        pbase = 2 * g * L
        plsc.store_scatter(pe_v, [pbase + 2 * iot], idx1)
        plsc.store_scatter(pe_v, [pbase + 2 * iot + 1], idx2)
        plsc.store_scatter(pwf_v, [pbase + 2 * iot], w1)
        plsc.store_scatter(pwf_v, [pbase + 2 * iot + 1], w2)
        for e in range(E):
            cnt[e] = cnt[e] + plsc.all_reduce_population_count(idx1 == e)
            cnt[e] = cnt[e] + plsc.all_reduce_population_count(idx2 == e)

    cnt_vec = jnp.zeros((L,), jnp.int32)
    for e in range(E):
        cnt_vec = cnt_vec + jnp.where(iot == e, cnt[e], 0)
    row16_v[...] = cnt_vec
    pltpu.sync_copy(row16_v, cnt_ref.at[s])
    pltpu.sync_copy(pe_v, pe_ref.at[s])
    pltpu.sync_copy(pwf_v, pw_ref.at[s])


def _route(rl_flat):
    mesh = plsc.VectorSubcoreMesh(core_axis_name="c", subcore_axis_name="s",
                                  num_cores=1)
    f = pl.kernel(
        _route_body,
        compiler_params=pltpu.CompilerParams(needs_layout_passes=False),
        out_type=(
            jax.ShapeDtypeStruct((D_TILES, L), jnp.int32),
            jax.ShapeDtypeStruct((D_TILES, PAIR_PER), jnp.int32),
            jax.ShapeDtypeStruct((D_TILES, PAIR_PER), jnp.float32),
        ),
        mesh=mesh,
        scratch_types=[
            pltpu.VMEM((TOK_PER * E,), jnp.float32),
            pltpu.VMEM((PAIR_PER,), jnp.int32),
            pltpu.VMEM((PAIR_PER,), jnp.float32),
            pltpu.VMEM((L,), jnp.int32),
        ],
    )
    return f(rl_flat)


def _sort_body(cnt_ref, pe_in_ref, pos_ref, tok_ref, te_ref,
               pe_v, row16_v, pos2_v, tok2_v, cnt2d_v):
    s = lax.axis_index("s")
    iot = _iota()

    pltpu.sync_copy(cnt_ref, cnt2d_v)
    pltpu.sync_copy(pe_in_ref.at[s], pe_v)
    tot = jnp.zeros((L,), jnp.int32)
    pre = jnp.zeros((L,), jnp.int32)
    s_spl = jnp.full((L,), s, jnp.int32)
    for t in range(D_TILES):
        row = cnt2d_v[t]
        pre = pre + jnp.where(_splat(t) < s_spl, row, 0)
        tot = tot + row
    rup = ((tot + (BM - 1)) >> 8) << 8
    off_incl = plsc.cumsum(rup)
    off_al = off_incl - rup
    start_vec = off_al + pre

    @pl.when(s == 0)
    def _():
        ends = [_bcast_lane(off_incl, e) for e in range(E - 1)]
        for c in range(NTE // L):
            pos0 = (c * L + iot) * BM
            te = jnp.zeros((L,), jnp.int32)
            for e in range(E - 1):
                te = te + jnp.where(pos0 >= ends[e], 1, 0)
            row16_v[...] = te
            pltpu.sync_copy(row16_v, te_ref.at[pl.ds(c * L, L)])

    start = [_bcast_lane(start_vec, e) for e in range(E)]
    for g in range(PAIR_PER // L):
        pe_g = pe_v[pl.ds(g * L, L)]
        pos = jnp.zeros((L,), jnp.int32)
        for e in range(E):
            m = pe_g == e
            incl = plsc.cumsum(jnp.where(m, 1, 0))
            pos = jnp.where(m, start[e] + incl - 1, pos)
            start[e] = start[e] + plsc.all_reduce_population_count(m)
        pos2_v[g] = pos
        tok2_v[g] = s_spl * TOK_PER + ((g * L + iot) >> 1)

    pltpu.sync_copy(pos2_v, pos_ref.at[s])
    pltpu.sync_copy(tok2_v, tok_ref.at[s])


def _sort(cnts, pe3):
    mesh = plsc.VectorSubcoreMesh(core_axis_name="c", subcore_axis_name="s",
                                  num_cores=1)
    f = pl.kernel(
        _sort_body,
        compiler_params=pltpu.CompilerParams(needs_layout_passes=False),
        out_type=(
            jax.ShapeDtypeStruct((D_TILES, PAIR_PER // L, L), jnp.int32),
            jax.ShapeDtypeStruct((D_TILES, PAIR_PER // L, L), jnp.int32),
            jax.ShapeDtypeStruct((NTE,), jnp.int32),
        ),
        mesh=mesh,
        scratch_types=[
            pltpu.VMEM((PAIR_PER,), jnp.int32),
            pltpu.VMEM((L,), jnp.int32),
            pltpu.VMEM((PAIR_PER // L, L), jnp.int32),
            pltpu.VMEM((PAIR_PER // L, L), jnp.int32),
            pltpu.VMEM((D_TILES, L), jnp.int32),
        ],
    )
    return f(cnts, pe3)


K2_CH = 16
K2_B = (P // NW) // K2_CH


def _shuffle_body(xb_ref, tok_ref, pos_ref, xs_ref,
                  tokv, posv, bufa, bufb, gsem, ssem):
    wid = lax.axis_index("s") * 2 + lax.axis_index("c")
    s = wid // 2
    h = wid % 2
    pltpu.sync_copy(tok_ref.at[s, pl.ds(h * K2_CH, K2_CH)], tokv)
    pltpu.sync_copy(pos_ref.at[s, pl.ds(h * K2_CH, K2_CH)], posv)
    prev = None
    for c in range(K2_CH):
        buf = bufa if c % 2 == 0 else bufb
        pltpu.async_copy(xb_ref.at[tokv.at[c]], buf, gsem).wait()
        if prev is not None:
            prev.wait()
        prev = pltpu.async_copy(buf, xs_ref.at[posv.at[c]], ssem)
    prev.wait()


def _shuffle(x, tok3, pos3):
    mesh = plsc.VectorSubcoreMesh(core_axis_name="c", subcore_axis_name="s")
    f = pl.kernel(
        _shuffle_body,
        compiler_params=pltpu.CompilerParams(needs_layout_passes=False),
        out_type=jax.ShapeDtypeStruct((C, D_MODEL), jnp.float32),
        mesh=mesh,
        scratch_types=[
            pltpu.VMEM((K2_CH, K2_B), jnp.int32),
            pltpu.VMEM((K2_CH, K2_B), jnp.int32),
            pltpu.VMEM((K2_B, D_MODEL), jnp.float32),
            pltpu.VMEM((K2_B, D_MODEL), jnp.float32),
            pltpu.SemaphoreType.DMA,
            pltpu.SemaphoreType.DMA,
        ],
    )
    return f(x, tok3, pos3)


def _gmm_body(te_ref, xs_ref, w13_ref, w2_ref, y_ref):
    gu = jnp.dot(xs_ref[...].astype(jnp.bfloat16), w13_ref[0],
                 preferred_element_type=jnp.float32)
    gate = gu[:, :D_FF]
    up = gu[:, D_FF:]
    h = (gate * jax.nn.sigmoid(gate)) * up
    y_ref[...] = jnp.dot(h.astype(jnp.bfloat16), w2_ref[0],
                         preferred_element_type=jnp.float32)


def _gmm(te, xs2, w13b, w2b):
    grid_spec = pltpu.PrefetchScalarGridSpec(
        num_scalar_prefetch=1,
        grid=(NT,),
        in_specs=[
            pl.BlockSpec((BM, D_MODEL), lambda t, te: (t, 0)),
            pl.BlockSpec((1, D_MODEL, 2 * D_FF), lambda t, te: (te[t], 0, 0)),
            pl.BlockSpec((1, D_FF, D_MODEL), lambda t, te: (te[t], 0, 0)),
        ],
        out_specs=pl.BlockSpec((BM, D_MODEL), lambda t, te: (t, 0)),
    )
    return pl.pallas_call(
        _gmm_body,
        grid_spec=grid_spec,
        out_shape=jax.ShapeDtypeStruct((C, D_MODEL), jnp.float32),
        compiler_params=pltpu.CompilerParams(
            dimension_semantics=("arbitrary",),
        ),
    )(te, xs2, w13b, w2b)


K4_CH = 8
K4_TOK = (T // NW) // K4_CH
K4_B = 2 * K4_TOK
LC = D_MODEL // L


def _combine_body(y_ref, pos_ref, pw_ref, out_ref, posv, pwv, buf, obuf, sem):
    wid = lax.axis_index("s") * 2 + lax.axis_index("c")
    s = wid // 2
    h = wid % 2
    pltpu.sync_copy(pos_ref.at[s, pl.ds(h * 16, 16)], posv)
    pltpu.sync_copy(pw_ref.at[s, pl.ds(h * (PAIR_PER // 2), PAIR_PER // 2)],
                    pwv)
    tbase = wid * (T // NW)
    for c in range(K4_CH):
        pltpu.async_copy(y_ref.at[posv.at[2 * c]],
                         buf.at[pl.ds(0, L)], sem).wait()
        pltpu.async_copy(y_ref.at[posv.at[2 * c + 1]],
                         buf.at[pl.ds(L, L)], sem).wait()
        wlo = pwv[pl.ds(c * 2 * L, L)]
        whi = pwv[pl.ds(c * 2 * L + L, L)]
        ws = ([_bcast_lane(wlo, i) for i in range(L)]
              + [_bcast_lane(whi, i) for i in range(L)])

        def body(j, _):
            for i in range(K4_TOK):
                a = buf[2 * i, pl.ds(j * L, L)]
                b = buf[2 * i + 1, pl.ds(j * L, L)]
                obuf[i, pl.ds(j * L, L)] = ws[2 * i] * a + ws[2 * i + 1] * b
            return 0

        lax.fori_loop(0, LC, body, 0)
        pltpu.sync_copy(obuf, out_ref.at[pl.ds(tbase + c * K4_TOK, K4_TOK)])


def _combine(y, pos3, pw3):
    mesh = plsc.VectorSubcoreMesh(core_axis_name="c", subcore_axis_name="s")
    f = pl.kernel(
        _combine_body,
        compiler_params=pltpu.CompilerParams(needs_layout_passes=False),
        out_type=jax.ShapeDtypeStruct((T, D_MODEL), jnp.float32),
        mesh=mesh,
        scratch_types=[
            pltpu.VMEM((16, L), jnp.int32),
            pltpu.VMEM((PAIR_PER // 2,), jnp.float32),
            pltpu.VMEM((K4_B, D_MODEL), jnp.float32),
            pltpu.VMEM((K4_TOK, D_MODEL), jnp.float32),
            pltpu.SemaphoreType.DMA,
        ],
    )
    return f(y, pos3, pw3)


def kernel(x, router_logits, w13, w2):
    rl_flat = router_logits.reshape(-1)
    w13b = w13.astype(jnp.bfloat16)
    w2b = w2.astype(jnp.bfloat16)

    cnts, pe3, pw2d = _route(rl_flat)
    pos, tok, te = _sort(cnts, pe3)

    xs = _shuffle(x, tok, pos)
    y = _gmm(te, xs, w13b, w2b)
    out = _combine(y, pos, pw2d)
    return out


# speedup vs baseline: 2.5008x; 1.0815x over previous
import functools

import jax
import jax.numpy as jnp
from jax import lax
from jax.experimental import pallas as pl
from jax.experimental.pallas import tpu as pltpu
from jax.experimental.pallas import tpu_sc as plsc

E = 8
TOP_K = 2
D_MODEL = 2048
D_FF = 1024
T = 4096
P = T * TOP_K
BM = 256
C = P + E * BM
NT = C // BM
NTE = 48
L = 16

D_TILES = 16
TOK_PER = T // D_TILES
PAIR_PER = 2 * TOK_PER
NW = 32


def _iota():
    return lax.iota(jnp.int32, L)


def _splat(val):
    return jnp.full((L,), val, jnp.int32)


def _bcast_lane(vec, lane):
    zero = jnp.zeros((), vec.dtype)
    return jnp.full((L,), jnp.sum(jnp.where(_iota() == lane, vec, zero)))


def _route_body(rl_ref, cnt_ref, pe_ref, pw_ref, lg_v, pe_v, pwf_v, row16_v):
    s = lax.axis_index("s")
    iot = _iota()

    pltpu.sync_copy(rl_ref.at[pl.ds(s * TOK_PER * E, TOK_PER * E)], lg_v)
    cnt = [jnp.zeros((L,), jnp.int32) for _ in range(E)]
    minf = jnp.full((L,), -jnp.inf, jnp.float32)
    for g in range(TOK_PER // L):
        base = g * L * E
        vals = [plsc.load_gather(lg_v, [base + iot * E + e]) for e in range(E)]
        m1 = vals[0]
        for e in range(1, E):
            m1 = jnp.maximum(m1, vals[e])
        idx1 = _splat(E)
        for e in range(E):
            idx1 = jnp.minimum(idx1, jnp.where(vals[e] == m1, e, E))
        m2 = minf
        for e in range(E):
            v2 = jnp.where(idx1 == e, minf, vals[e])
            m2 = jnp.maximum(m2, v2)
        idx2 = _splat(E)
        for e in range(E):
            v2 = jnp.where(idx1 == e, minf, vals[e])
            idx2 = jnp.minimum(idx2, jnp.where(v2 == m2, e, E))
        w1 = 1.0 / (1.0 + jnp.exp(m2 - m1))
        w2 = 1.0 - w1
        pbase = 2 * g * L
        plsc.store_scatter(pe_v, [pbase + 2 * iot], idx1)
        plsc.store_scatter(pe_v, [pbase + 2 * iot + 1], idx2)
        plsc.store_scatter(pwf_v, [pbase + 2 * iot], w1)
        plsc.store_scatter(pwf_v, [pbase + 2 * iot + 1], w2)
        for e in range(E):
            cnt[e] = cnt[e] + plsc.all_reduce_population_count(idx1 == e)
            cnt[e] = cnt[e] + plsc.all_reduce_population_count(idx2 == e)

    cnt_vec = jnp.zeros((L,), jnp.int32)
    for e in range(E):
        cnt_vec = cnt_vec + jnp.where(iot == e, cnt[e], 0)
    row16_v[...] = cnt_vec
    pltpu.sync_copy(row16_v, cnt_ref.at[s])
    pltpu.sync_copy(pe_v, pe_ref.at[s])
    pltpu.sync_copy(pwf_v, pw_ref.at[s])


def _route(rl_flat):
    mesh = plsc.VectorSubcoreMesh(core_axis_name="c", subcore_axis_name="s",
                                  num_cores=1)
    f = pl.kernel(
        _route_body,
        compiler_params=pltpu.CompilerParams(needs_layout_passes=False),
        out_type=(
            jax.ShapeDtypeStruct((D_TILES, L), jnp.int32),
            jax.ShapeDtypeStruct((D_TILES, PAIR_PER), jnp.int32),
            jax.ShapeDtypeStruct((D_TILES, PAIR_PER), jnp.float32),
        ),
        mesh=mesh,
        scratch_types=[
            pltpu.VMEM((TOK_PER * E,), jnp.float32),
            pltpu.VMEM((PAIR_PER,), jnp.int32),
            pltpu.VMEM((PAIR_PER,), jnp.float32),
            pltpu.VMEM((L,), jnp.int32),
        ],
    )
    return f(rl_flat)


def _sort_body(cnt_ref, pe_in_ref, pos_ref, tok_ref, te_ref,
               pe_v, row16_v, pos2_v, tok2_v, cnt2d_v):
    s = lax.axis_index("s")
    iot = _iota()

    pltpu.sync_copy(cnt_ref, cnt2d_v)
    pltpu.sync_copy(pe_in_ref.at[s], pe_v)
    tot = jnp.zeros((L,), jnp.int32)
    pre = jnp.zeros((L,), jnp.int32)
    s_spl = jnp.full((L,), s, jnp.int32)
    for t in range(D_TILES):
        row = cnt2d_v[t]
        pre = pre + jnp.where(_splat(t) < s_spl, row, 0)
        tot = tot + row
    rup = ((tot + (BM - 1)) >> 8) << 8
    off_incl = plsc.cumsum(rup)
    off_al = off_incl - rup
    start_vec = off_al + pre

    @pl.when(s == 0)
    def _():
        ends = [_bcast_lane(off_incl, e) for e in range(E - 1)]
        for c in range(NTE // L):
            pos0 = (c * L + iot) * BM
            te = jnp.zeros((L,), jnp.int32)
            for e in range(E - 1):
                te = te + jnp.where(pos0 >= ends[e], 1, 0)
            row16_v[...] = te
            pltpu.sync_copy(row16_v, te_ref.at[pl.ds(c * L, L)])

    start = [_bcast_lane(start_vec, e) for e in range(E)]
    for g in range(PAIR_PER // L):
        pe_g = pe_v[pl.ds(g * L, L)]
        pos = jnp.zeros((L,), jnp.int32)
        for e in range(E):
            m = pe_g == e
            incl = plsc.cumsum(jnp.where(m, 1, 0))
            pos = jnp.where(m, start[e] + incl - 1, pos)
            start[e] = start[e] + plsc.all_reduce_population_count(m)
        pos2_v[g] = pos
        tok2_v[g] = s_spl * TOK_PER + ((g * L + iot) >> 1)

    pltpu.sync_copy(pos2_v, pos_ref.at[s])
    pltpu.sync_copy(tok2_v, tok_ref.at[s])


def _sort(cnts, pe3):
    mesh = plsc.VectorSubcoreMesh(core_axis_name="c", subcore_axis_name="s",
                                  num_cores=1)
    f = pl.kernel(
        _sort_body,
        compiler_params=pltpu.CompilerParams(needs_layout_passes=False),
        out_type=(
            jax.ShapeDtypeStruct((D_TILES, PAIR_PER // L, L), jnp.int32),
            jax.ShapeDtypeStruct((D_TILES, PAIR_PER // L, L), jnp.int32),
            jax.ShapeDtypeStruct((NTE,), jnp.int32),
        ),
        mesh=mesh,
        scratch_types=[
            pltpu.VMEM((PAIR_PER,), jnp.int32),
            pltpu.VMEM((L,), jnp.int32),
            pltpu.VMEM((PAIR_PER // L, L), jnp.int32),
            pltpu.VMEM((PAIR_PER // L, L), jnp.int32),
            pltpu.VMEM((D_TILES, L), jnp.int32),
        ],
    )
    return f(cnts, pe3)


DH = D_MODEL // 2


def _rne_bf16_bits(f):
    u = jax.lax.bitcast_convert_type(f, jnp.uint32)
    lsb = (u >> 16) & jnp.uint32(1)
    return (u + jnp.uint32(0x7FFF) + lsb) >> 16


def _pack_halves(f):
    r = _rne_bf16_bits(f)
    packed = r[:, :DH] | (r[:, DH:] << 16)
    return jax.lax.bitcast_convert_type(packed, jnp.int32)


def _unpack_halves_bf16(v):
    lo = jax.lax.bitcast_convert_type(v << 16, jnp.float32)
    hi = jax.lax.bitcast_convert_type(v & jnp.int32(-65536), jnp.float32)
    return jnp.concatenate([lo, hi], axis=1).astype(jnp.bfloat16)


def _pack_body(x_ref, px_ref):
    px_ref[...] = _pack_halves(x_ref[...])


def _pack(x):
    return pl.pallas_call(
        _pack_body,
        grid=(T // BM,),
        in_specs=[pl.BlockSpec((BM, D_MODEL), lambda t: (t, 0))],
        out_specs=pl.BlockSpec((BM, DH), lambda t: (t, 0)),
        out_shape=jax.ShapeDtypeStruct((T, DH), jnp.int32),
        compiler_params=pltpu.CompilerParams(
            dimension_semantics=("parallel",),
        ),
    )(x)


K2_CH = 16
K2_B = (P // NW) // K2_CH


def _shuffle_body(xb_ref, tok_ref, pos_ref, xs_ref,
                  tokv, posv, bufa, bufb, gsem, ssem):
    wid = lax.axis_index("s") * 2 + lax.axis_index("c")
    s = wid // 2
    h = wid % 2
    pltpu.sync_copy(tok_ref.at[s, pl.ds(h * K2_CH, K2_CH)], tokv)
    pltpu.sync_copy(pos_ref.at[s, pl.ds(h * K2_CH, K2_CH)], posv)
    prev = None
    for c in range(K2_CH):
        buf = bufa if c % 2 == 0 else bufb
        pltpu.async_copy(xb_ref.at[tokv.at[c]], buf, gsem).wait()
        if prev is not None:
            prev.wait()
        prev = pltpu.async_copy(buf, xs_ref.at[posv.at[c]], ssem)
    prev.wait()


def _shuffle(x, tok3, pos3):
    mesh = plsc.VectorSubcoreMesh(core_axis_name="c", subcore_axis_name="s")
    f = pl.kernel(
        _shuffle_body,
        compiler_params=pltpu.CompilerParams(needs_layout_passes=False),
        out_type=jax.ShapeDtypeStruct((C, DH), jnp.int32),
        mesh=mesh,
        scratch_types=[
            pltpu.VMEM((K2_CH, K2_B), jnp.int32),
            pltpu.VMEM((K2_CH, K2_B), jnp.int32),
            pltpu.VMEM((K2_B, DH), jnp.int32),
            pltpu.VMEM((K2_B, DH), jnp.int32),
            pltpu.SemaphoreType.DMA,
            pltpu.SemaphoreType.DMA,
        ],
    )
    return f(x, tok3, pos3)


def _gmm_body(te_ref, xs_ref, w13_ref, w2_ref, y_ref):
    xb = _unpack_halves_bf16(xs_ref[...])
    gu = jnp.dot(xb, w13_ref[0], preferred_element_type=jnp.float32)
    gate = gu[:, :D_FF]
    up = gu[:, D_FF:]
    h = (gate * jax.nn.sigmoid(gate)) * up
    y = jnp.dot(h.astype(jnp.bfloat16), w2_ref[0],
                preferred_element_type=jnp.float32)
    y_ref[...] = _pack_halves(y)


def _gmm(te, xs2, w13b, w2b):
    grid_spec = pltpu.PrefetchScalarGridSpec(
        num_scalar_prefetch=1,
        grid=(NT,),
        in_specs=[
            pl.BlockSpec((BM, DH), lambda t, te: (t, 0)),
            pl.BlockSpec((1, D_MODEL, 2 * D_FF), lambda t, te: (te[t], 0, 0)),
            pl.BlockSpec((1, D_FF, D_MODEL), lambda t, te: (te[t], 0, 0)),
        ],
        out_specs=pl.BlockSpec((BM, DH), lambda t, te: (t, 0)),
    )
    return pl.pallas_call(
        _gmm_body,
        grid_spec=grid_spec,
        out_shape=jax.ShapeDtypeStruct((C, DH), jnp.int32),
        compiler_params=pltpu.CompilerParams(
            dimension_semantics=("arbitrary",),
        ),
    )(te, xs2, w13b, w2b)


K4_CH = 8
K4_TOK = (T // NW) // K4_CH
K4_B = 2 * K4_TOK
LC = D_MODEL // L


def _combine_body(y_ref, pos_ref, pw_ref, out_ref, posv, pwv, buf, obuf, sem):
    wid = lax.axis_index("s") * 2 + lax.axis_index("c")
    s = wid // 2
    h = wid % 2
    pltpu.sync_copy(pos_ref.at[s, pl.ds(h * 16, 16)], posv)
    pltpu.sync_copy(pw_ref.at[s, pl.ds(h * (PAIR_PER // 2), PAIR_PER // 2)],
                    pwv)
    tbase = wid * (T // NW)
    for c in range(K4_CH):
        pltpu.async_copy(y_ref.at[posv.at[2 * c]],
                         buf.at[pl.ds(0, L)], sem).wait()
        pltpu.async_copy(y_ref.at[posv.at[2 * c + 1]],
                         buf.at[pl.ds(L, L)], sem).wait()
        wlo = pwv[pl.ds(c * 2 * L, L)]
        whi = pwv[pl.ds(c * 2 * L + L, L)]
        ws = ([_bcast_lane(wlo, i) for i in range(L)]
              + [_bcast_lane(whi, i) for i in range(L)])

        def body(j, _):
            for i in range(K4_TOK):
                va = buf[2 * i, pl.ds(j * L, L)]
                vb = buf[2 * i + 1, pl.ds(j * L, L)]
                lo_a = plsc.bitcast(va << 16, jnp.float32)
                hi_a = plsc.bitcast(va & jnp.int32(-65536), jnp.float32)
                lo_b = plsc.bitcast(vb << 16, jnp.float32)
                hi_b = plsc.bitcast(vb & jnp.int32(-65536), jnp.float32)
                wa = ws[2 * i]
                wb = ws[2 * i + 1]
                obuf[i, pl.ds(j * L, L)] = wa * lo_a + wb * lo_b
                obuf[i, pl.ds(DH + j * L, L)] = wa * hi_a + wb * hi_b
            return 0

        lax.fori_loop(0, DH // L, body, 0)
        pltpu.sync_copy(obuf, out_ref.at[pl.ds(tbase + c * K4_TOK, K4_TOK)])


def _combine(y, pos3, pw3):
    mesh = plsc.VectorSubcoreMesh(core_axis_name="c", subcore_axis_name="s")
    f = pl.kernel(
        _combine_body,
        compiler_params=pltpu.CompilerParams(needs_layout_passes=False),
        out_type=jax.ShapeDtypeStruct((T, D_MODEL), jnp.float32),
        mesh=mesh,
        scratch_types=[
            pltpu.VMEM((16, L), jnp.int32),
            pltpu.VMEM((PAIR_PER // 2,), jnp.float32),
            pltpu.VMEM((K4_B, DH), jnp.int32),
            pltpu.VMEM((K4_TOK, D_MODEL), jnp.float32),
            pltpu.SemaphoreType.DMA,
        ],
    )
    return f(y, pos3, pw3)


def kernel(x, router_logits, w13, w2):
    rl_flat = router_logits.reshape(-1)
    w13b = w13.astype(jnp.bfloat16)
    w2b = w2.astype(jnp.bfloat16)

    cnts, pe3, pw2d = _route(rl_flat)
    pos, tok, te = _sort(cnts, pe3)

    px = _pack(x)
    xs = _shuffle(px, tok, pos)
    y = _gmm(te, xs, w13b, w2b)
    out = _combine(y, pos, pw2d)
    return out
